# Initial kernel scaffold; baseline (speedup 1.0000x reference)
import jax
import jax.numpy as jnp
from jax.experimental import pallas as pl


def kernel(cart, lats, senders, receivers, to_jimage, edge_graph_i, species, W_proj, b_proj, species_table):
    raise NotImplementedError("write your pallas kernel here")


# speedup vs baseline: 6.1851x; 6.1851x over previous
import jax
import jax.numpy as jnp
from jax import lax
from jax.experimental import pallas as pl
from jax.experimental.pallas import tpu as pltpu
from jax.experimental.pallas import tpu_sc as plsc

N_NODES = 50000
N_EDGES = 800000
N_GRAPHS = 128
NODE_EMB = 128
N_RBF = 32
EDGE_EMB = 32
CUTOFF = 6.0
SIGMA = CUTOFF / N_RBF
INV2S2 = 1.0 / (2.0 * SIGMA * SIGMA)

NC, NS = 2, 16
NW = NC * NS
EPW = N_EDGES // NW
ECHUNK = 1000
NECHUNK = EPW // ECHUNK
NODE_WORKERS = 25
NPW = N_NODES // NODE_WORKERS


def _sc_body(cartx, carty, cartz, senders, receivers, jx, jy, jz, gidx,
             lats_flat, species, table, vecs_out, nemb_out,
             sidx, ridx, sxb, syb, szb, rxb, ryb, rzb,
             jxb, jyb, jzb, gb, latb, vb, spid, nrows,
             cxsh, cysh, czsh, sem_in, sem_g, sem_n):
    sid = lax.axis_index("s")
    wid = sid * NC + lax.axis_index("c")

    pltpu.sync_copy(lats_flat, latb)

    @pl.when(sid == 0)
    def _():
        pltpu.sync_copy(cartx, cxsh)
        pltpu.sync_copy(carty, cysh)
        pltpu.sync_copy(cartz, czsh)
    plsc.subcore_barrier()

    iot = lax.iota(jnp.int32, 16)

    def edge_group(o):
        rows = o + iot
        sx = sxb[pl.ds(o, 16)]
        sy = syb[pl.ds(o, 16)]
        sz = szb[pl.ds(o, 16)]
        rx = rxb[pl.ds(o, 16)]
        ry = ryb[pl.ds(o, 16)]
        rz = rzb[pl.ds(o, 16)]
        g9 = gb[pl.ds(o, 16)] * 9
        ja = jxb[pl.ds(o, 16)]
        jb_ = jyb[pl.ds(o, 16)]
        jc = jzb[pl.ds(o, 16)]
        l00 = plsc.load_gather(latb, [g9])
        l01 = plsc.load_gather(latb, [g9 + 1])
        l02 = plsc.load_gather(latb, [g9 + 2])
        l10 = plsc.load_gather(latb, [g9 + 3])
        l11 = plsc.load_gather(latb, [g9 + 4])
        l12 = plsc.load_gather(latb, [g9 + 5])
        l20 = plsc.load_gather(latb, [g9 + 6])
        l21 = plsc.load_gather(latb, [g9 + 7])
        l22 = plsc.load_gather(latb, [g9 + 8])
        vx = rx + (l00 * ja + l10 * jb_ + l20 * jc) - sx
        vy = ry + (l01 * ja + l11 * jb_ + l21 * jc) - sy
        vz = rz + (l02 * ja + l12 * jb_ + l22 * jc) - sz
        r3 = rows * 3
        plsc.store_scatter(vb, [r3], vx)
        plsc.store_scatter(vb, [r3 + 1], vy)
        plsc.store_scatter(vb, [r3 + 2], vz)

    def chunk_body(ci, carry):
        base = wid * EPW + ci * ECHUNK
        cps = []
        for src, dst in ((senders, sidx), (receivers, ridx), (jx, jxb),
                         (jy, jyb), (jz, jzb), (gidx, gb)):
            cps.append(pltpu.async_copy(src.at[pl.ds(base, ECHUNK)], dst,
                                        sem_in))
        for cp in cps:
            cp.wait()
        gcps = []
        for j in range(8):
            n = 128 if j < 7 else ECHUNK - 7 * 128
            sl = pl.ds(j * 128, n)
            for tab, idx, dst in ((cxsh, sidx, sxb), (cysh, sidx, syb),
                                  (czsh, sidx, szb), (cxsh, ridx, rxb),
                                  (cysh, ridx, ryb), (czsh, ridx, rzb)):
                gcps.append(pltpu.async_copy(tab.at[idx.at[sl]], dst.at[sl],
                                             sem_g))
        for cp in gcps:
            cp.wait()

        def g_body(i, c):
            edge_group(i * 16)
            return c
        lax.fori_loop(0, ECHUNK // 16, g_body, 0)
        edge_group(ECHUNK - 16)
        pltpu.sync_copy(vb, vecs_out.at[pl.ds(3 * base, 3 * ECHUNK)])
        return carry

    lax.fori_loop(0, NECHUNK, chunk_body, 0)

    @pl.when(wid < NODE_WORKERS)
    def _():
        nb = wid * NPW
        for j in range(16):
            n = 128 if j < 15 else NPW - 15 * 128
            sl = pl.ds(0, n)
            pltpu.sync_copy(species.at[pl.ds(nb + j * 128, n)], spid.at[sl])
            pltpu.async_copy(table.at[spid.at[sl]], nrows.at[sl],
                             sem_n).wait()
            pltpu.sync_copy(nrows.at[sl],
                            nemb_out.at[pl.ds(nb + j * 128, n)])


_sc_call = pl.kernel(
    _sc_body,
    out_type=[
        jax.ShapeDtypeStruct((3 * N_EDGES,), jnp.float32),
        jax.ShapeDtypeStruct((N_NODES, NODE_EMB), jnp.float32),
    ],
    mesh=plsc.VectorSubcoreMesh(core_axis_name="c", subcore_axis_name="s"),
    compiler_params=pltpu.CompilerParams(needs_layout_passes=False,
                                         use_tc_tiling_on_sc=False),
    scratch_types=[
        pltpu.VMEM((ECHUNK,), jnp.int32),
        pltpu.VMEM((ECHUNK,), jnp.int32),
        pltpu.VMEM((ECHUNK,), jnp.float32),
        pltpu.VMEM((ECHUNK,), jnp.float32),
        pltpu.VMEM((ECHUNK,), jnp.float32),
        pltpu.VMEM((ECHUNK,), jnp.float32),
        pltpu.VMEM((ECHUNK,), jnp.float32),
        pltpu.VMEM((ECHUNK,), jnp.float32),
        pltpu.VMEM((ECHUNK,), jnp.float32),
        pltpu.VMEM((ECHUNK,), jnp.float32),
        pltpu.VMEM((ECHUNK,), jnp.float32),
        pltpu.VMEM((ECHUNK,), jnp.int32),
        pltpu.VMEM((N_GRAPHS * 9,), jnp.float32),
        pltpu.VMEM((3 * ECHUNK,), jnp.float32),
        pltpu.VMEM((128,), jnp.int32),
        pltpu.VMEM((128, NODE_EMB), jnp.float32),
        pltpu.VMEM_SHARED((N_NODES,), jnp.float32),
        pltpu.VMEM_SHARED((N_NODES,), jnp.float32),
        pltpu.VMEM_SHARED((N_NODES,), jnp.float32),
        pltpu.SemaphoreType.DMA,
        pltpu.SemaphoreType.DMA,
        pltpu.SemaphoreType.DMA,
    ],
)

RB = 2000


def _tc_body(vecs_ref, w_ref, b_ref, dist_ref, emb_ref):
    v = vecs_ref[...]
    d2 = jnp.sum(v * v, axis=1, keepdims=True) + 1e-12
    d = jnp.sqrt(d2)
    dist_ref[...] = d
    mu = lax.broadcasted_iota(jnp.int32, (1, N_RBF), 1).astype(
        jnp.float32) * (CUTOFF / (N_RBF - 1))
    delta = d - mu
    rbf = jnp.exp(-(delta * delta) * INV2S2)
    emb_ref[...] = jnp.dot(rbf, w_ref[...],
                           preferred_element_type=jnp.float32) + b_ref[...]


_tc_call = pl.pallas_call(
    _tc_body,
    grid=(N_EDGES // RB,),
    in_specs=[
        pl.BlockSpec((RB, 3), lambda i: (i, 0)),
        pl.BlockSpec((N_RBF, EDGE_EMB), lambda i: (0, 0)),
        pl.BlockSpec((1, EDGE_EMB), lambda i: (0, 0)),
    ],
    out_specs=[
        pl.BlockSpec((RB, 1), lambda i: (i, 0)),
        pl.BlockSpec((RB, EDGE_EMB), lambda i: (i, 0)),
    ],
    out_shape=[
        jax.ShapeDtypeStruct((N_EDGES, 1), jnp.float32),
        jax.ShapeDtypeStruct((N_EDGES, EDGE_EMB), jnp.float32),
    ],
)


def kernel(cart, lats, senders, receivers, to_jimage, edge_graph_i, species,
           W_proj, b_proj, species_table):
    cartx, carty, cartz = cart[:, 0], cart[:, 1], cart[:, 2]
    jimf = to_jimage.astype(jnp.float32)
    jx, jy, jz = jimf[:, 0], jimf[:, 1], jimf[:, 2]
    lats_flat = lax.reduce_precision(lats, exponent_bits=8,
                                     mantissa_bits=7).reshape(-1)
    vecs_flat, node_emb = _sc_call(cartx, carty, cartz, senders, receivers,
                                   jx, jy, jz, edge_graph_i, lats_flat,
                                   species, species_table)
    vecs = vecs_flat.reshape(N_EDGES, 3)
    dist2d, edge_emb = _tc_call(vecs, W_proj, b_proj.reshape(1, EDGE_EMB))
    return node_emb, edge_emb, vecs, dist2d.reshape(N_EDGES)
